# Initial kernel scaffold; baseline (speedup 1.0000x reference)
import jax
import jax.numpy as jnp
from jax.experimental import pallas as pl


def kernel(x, weight, rel_weight, nodes, neighbors, relations):
    raise NotImplementedError("write your pallas kernel here")


# speedup vs baseline: 2.9871x; 2.9871x over previous
import functools

import jax
import jax.numpy as jnp
from jax import lax
from jax.experimental import pallas as pl
from jax.experimental.pallas import tpu as pltpu
from jax.experimental.pallas import tpu_sc as plsc

_NC = 2
_NS = 16
_NW = _NC * _NS
_LANES = 16


def _sc_gather_agg(x, nodes, nb_flat, dest, *, B, S, R, D, CB):
    rows_w = B // _NW
    nch = rows_w // CB
    gw = 64
    ng = (CB * S) // gw
    segs = D // _LANES

    mesh = plsc.VectorSubcoreMesh(core_axis_name="c", subcore_axis_name="s")

    @functools.partial(
        pl.kernel,
        out_type=(
            jax.ShapeDtypeStruct((B, D), jnp.float32),
            jax.ShapeDtypeStruct((B * R, D), jnp.float32),
        ),
        mesh=mesh,
        scratch_types=[
            pltpu.VMEM((CB * S,), jnp.int32),
            pltpu.VMEM((CB,), jnp.int32),
            pltpu.VMEM((CB * S, D), jnp.float32),
            pltpu.VMEM((CB, D), jnp.float32),
            pltpu.VMEM((CB, D), jnp.float32),
            pltpu.VMEM_SHARED((_NS * CB * R, D), jnp.float32),
            pltpu.VMEM((CB * S,), jnp.int32),
            pltpu.SemaphoreType.DMA,
        ],
    )
    def k(x_hbm, nodes_hbm, nb_hbm, dest_hbm, self_hbm, agg_hbm,
          nb_idx, s_idx, gbuf, sbuf, zbuf, abuf_sh, dstage, sem):
        sid = lax.axis_index("s")
        wid = sid * _NC + lax.axis_index("c")
        wbase = wid * rows_w
        rbase = sid * (CB * R)

        @pl.loop(0, CB)
        def _z(i):
            for kk in range(segs):
                zbuf[i, pl.ds(kk * _LANES, _LANES)] = jnp.zeros(
                    (_LANES,), jnp.float32)

        @pl.loop(0, nch)
        def _chunk(c):
            base = wbase + c * CB
            ebase = base * S
            pltpu.sync_copy(nb_hbm.at[pl.ds(ebase, CB * S)], nb_idx)
            pltpu.sync_copy(nodes_hbm.at[pl.ds(base, CB)], s_idx)
            pltpu.sync_copy(dest_hbm.at[pl.ds(ebase, CB * S)], dstage)

            for g in range(ng):
                pltpu.async_copy(x_hbm.at[nb_idx.at[pl.ds(g * gw, gw)]],
                                 gbuf.at[pl.ds(g * gw, gw)], sem).wait()
            pltpu.async_copy(x_hbm.at[s_idx], sbuf, sem).wait()

            for z in range(R):
                pltpu.sync_copy(zbuf, abuf_sh.at[pl.ds(rbase + z * CB, CB)])

            pltpu.sync_copy(gbuf, abuf_sh.at[dstage], add=True)

            pltpu.sync_copy(sbuf, self_hbm.at[pl.ds(base, CB)])
            pltpu.sync_copy(abuf_sh.at[pl.ds(rbase, CB * R)],
                            agg_hbm.at[pl.ds(base * R, CB * R)])

    return k(x, nodes, nb_flat, dest)


def _tc_combine(self_emb, agg, relations, weight, rel_weight,
                *, B, S, R, D, DOUT, BB):
    def body(self_ref, agg_ref, rel_ref, w_ref, rw_ref, out_ref):
        acc = lax.dot_general(self_ref[...], w_ref[...],
                              (((1,), (1,)), ((), ())),
                              preferred_element_type=jnp.float32)
        rel = rel_ref[...]
        for r in range(R):
            cnt = jnp.sum((rel == r).astype(jnp.float32), axis=1,
                          keepdims=True)
            a = agg_ref[:, r * D:(r + 1) * D] * (1.0 / (cnt + 1e-10))
            acc = acc + lax.dot_general(a, rw_ref[r],
                                        (((1,), (1,)), ((), ())),
                                        preferred_element_type=jnp.float32)
        out_ref[...] = jnp.maximum(acc, 0.0)

    return pl.pallas_call(
        body,
        grid=(B // BB,),
        in_specs=[
            pl.BlockSpec((BB, D), lambda i: (i, 0)),
            pl.BlockSpec((BB, R * D), lambda i: (i, 0)),
            pl.BlockSpec((BB, S), lambda i: (i, 0)),
            pl.BlockSpec((DOUT, D), lambda i: (0, 0)),
            pl.BlockSpec((R, DOUT, D), lambda i: (0, 0, 0)),
        ],
        out_specs=pl.BlockSpec((BB, DOUT), lambda i: (i, 0)),
        out_shape=jax.ShapeDtypeStruct((B, DOUT), jnp.float32),
    )(self_emb, agg, relations, weight, rel_weight)


def kernel(x, weight, rel_weight, nodes, neighbors, relations):
    N, D = x.shape
    B, S = neighbors.shape
    R = rel_weight.shape[0]
    DOUT = weight.shape[0]
    CB = 32

    nodes = nodes.astype(jnp.int32)
    nb_flat = neighbors.astype(jnp.int32).reshape(B * S)
    rel = relations.astype(jnp.int32)
    rows_w = B // _NW
    barange = jnp.arange(B, dtype=jnp.int32)
    sub = (barange // rows_w) // _NC
    dest = ((sub * (CB * R) + (barange % CB) * R)[:, None]
            + rel).reshape(B * S)

    self_emb, agg = _sc_gather_agg(x, nodes, nb_flat, dest,
                                   B=B, S=S, R=R, D=D, CB=CB)
    return _tc_combine(self_emb, agg.reshape(B, R * D), rel, weight,
                       rel_weight, B=B, S=S, R=R, D=D, DOUT=DOUT, BB=1024)


# speedup vs baseline: 4.0966x; 1.3714x over previous
import functools

import jax
import jax.numpy as jnp
from jax import lax
from jax.experimental import pallas as pl
from jax.experimental.pallas import tpu as pltpu
from jax.experimental.pallas import tpu_sc as plsc

_NC = 2
_NS = 16
_NW = _NC * _NS
_LANES = 16


def _sc_gather_agg(x, nodes, nb_flat, dest, *, B, S, R, D, CB):
    rows_w = B // _NW
    nch = rows_w // CB
    segs = D // _LANES
    windows = []
    off = 0
    while off < CB * S:
        w = min(128, CB * S - off)
        windows.append((off, w))
        off += w

    mesh = plsc.VectorSubcoreMesh(core_axis_name="c", subcore_axis_name="s")

    @functools.partial(
        pl.kernel,
        out_type=(
            jax.ShapeDtypeStruct((B, D), jnp.float32),
            jax.ShapeDtypeStruct((B * R, D), jnp.float32),
        ),
        mesh=mesh,
        scratch_types=[
            pltpu.VMEM((CB * S,), jnp.int32),
            pltpu.VMEM((CB * S,), jnp.int32),
            pltpu.VMEM((CB,), jnp.int32),
            pltpu.VMEM((CB,), jnp.int32),
            pltpu.VMEM((CB * S,), jnp.int32),
            pltpu.VMEM((CB * S,), jnp.int32),
            pltpu.VMEM((CB * S, D), jnp.float32),
            pltpu.VMEM((CB * S, D), jnp.float32),
            pltpu.VMEM((CB, D), jnp.float32),
            pltpu.VMEM((CB, D), jnp.float32),
            pltpu.VMEM((CB * R, D), jnp.float32),
            pltpu.VMEM_SHARED((_NS * CB * R, D), jnp.float32),
            pltpu.SemaphoreType.DMA,
            pltpu.SemaphoreType.DMA,
            pltpu.SemaphoreType.DMA,
            pltpu.SemaphoreType.DMA,
            pltpu.SemaphoreType.DMA,
        ],
    )
    def k(x_hbm, nodes_hbm, nb_hbm, dest_hbm, self_hbm, agg_hbm,
          nb_idx0, nb_idx1, s_idx0, s_idx1, dst0, dst1, gbuf0, gbuf1,
          sbuf0, sbuf1, zbuf, abuf_sh, sem_i0, sem_i1, sem_g0, sem_g1,
          sem_f):
        nb_idx = (nb_idx0, nb_idx1)
        s_idx = (s_idx0, s_idx1)
        dst = (dst0, dst1)
        gbuf = (gbuf0, gbuf1)
        sbuf = (sbuf0, sbuf1)
        sem_i = (sem_i0, sem_i1)
        sem_g = (sem_g0, sem_g1)

        sid = lax.axis_index("s")
        wid = sid * _NC + lax.axis_index("c")
        wbase = wid * rows_w
        rbase = sid * (CB * R)

        @pl.loop(0, CB * R)
        def _z(i):
            for kk in range(segs):
                zbuf[i, pl.ds(kk * _LANES, _LANES)] = jnp.zeros(
                    (_LANES,), jnp.float32)

        idx_descs = {}
        g_descs = {}
        f_descs = {}

        def start_idx(c):
            p = c % 2
            base = wbase + c * CB
            ebase = base * S
            idx_descs[c] = (
                pltpu.async_copy(nb_hbm.at[pl.ds(ebase, CB * S)],
                                 nb_idx[p], sem_i[p]),
                pltpu.async_copy(nodes_hbm.at[pl.ds(base, CB)],
                                 s_idx[p], sem_i[p]),
                pltpu.async_copy(dest_hbm.at[pl.ds(ebase, CB * S)],
                                 dst[p], sem_i[p]),
            )

        def start_gathers(c):
            p = c % 2
            descs = [pltpu.async_copy(x_hbm.at[s_idx[p]], sbuf[p], sem_g[p])]
            for (o, w) in windows:
                descs.append(pltpu.async_copy(
                    x_hbm.at[nb_idx[p].at[pl.ds(o, w)]],
                    gbuf[p].at[pl.ds(o, w)], sem_g[p]))
            g_descs[c] = descs

        start_idx(0)
        for d in idx_descs[0]:
            d.wait()
        start_gathers(0)
        if nch > 1:
            start_idx(1)

        for c in range(nch):
            p = c % 2
            base = wbase + c * CB
            for d in g_descs[c]:
                d.wait()

            if c > 0:
                f_descs[c - 1].wait()
            pltpu.sync_copy(zbuf, abuf_sh.at[pl.ds(rbase, CB * R)])
            pltpu.sync_copy(gbuf[p], abuf_sh.at[dst[p]], add=True)
            if c + 2 < nch:
                start_idx(c + 2)
            if c + 1 < nch:
                for d in idx_descs[c + 1]:
                    d.wait()
                start_gathers(c + 1)
            f_descs[c] = pltpu.async_copy(
                abuf_sh.at[pl.ds(rbase, CB * R)],
                agg_hbm.at[pl.ds(base * R, CB * R)], sem_f)
            pltpu.sync_copy(sbuf[p], self_hbm.at[pl.ds(base, CB)])
        f_descs[nch - 1].wait()

    return k(x, nodes, nb_flat, dest)


def _tc_combine(self_emb, agg, relations, weight, rel_weight,
                *, B, S, R, D, DOUT, BB):
    def body(self_ref, agg_ref, rel_ref, w_ref, rw_ref, out_ref):
        acc = lax.dot_general(self_ref[...], w_ref[...],
                              (((1,), (1,)), ((), ())),
                              preferred_element_type=jnp.float32)
        rel = rel_ref[...]
        for r in range(R):
            cnt = jnp.sum((rel == r).astype(jnp.float32), axis=1,
                          keepdims=True)
            a = agg_ref[:, r * D:(r + 1) * D] * (1.0 / (cnt + 1e-10))
            acc = acc + lax.dot_general(a, rw_ref[r],
                                        (((1,), (1,)), ((), ())),
                                        preferred_element_type=jnp.float32)
        out_ref[...] = jnp.maximum(acc, 0.0)

    return pl.pallas_call(
        body,
        grid=(B // BB,),
        in_specs=[
            pl.BlockSpec((BB, D), lambda i: (i, 0)),
            pl.BlockSpec((BB, R * D), lambda i: (i, 0)),
            pl.BlockSpec((BB, S), lambda i: (i, 0)),
            pl.BlockSpec((DOUT, D), lambda i: (0, 0)),
            pl.BlockSpec((R, DOUT, D), lambda i: (0, 0, 0)),
        ],
        out_specs=pl.BlockSpec((BB, DOUT), lambda i: (i, 0)),
        out_shape=jax.ShapeDtypeStruct((B, DOUT), jnp.float32),
    )(self_emb, agg, relations, weight, rel_weight)


def kernel(x, weight, rel_weight, nodes, neighbors, relations):
    N, D = x.shape
    B, S = neighbors.shape
    R = rel_weight.shape[0]
    DOUT = weight.shape[0]
    CB = 32

    nodes = nodes.astype(jnp.int32)
    nb_flat = neighbors.astype(jnp.int32).reshape(B * S)
    rel = relations.astype(jnp.int32)
    rows_w = B // _NW
    barange = jnp.arange(B, dtype=jnp.int32)
    sub = (barange // rows_w) // _NC
    dest = ((sub * (CB * R) + (barange % CB) * R)[:, None]
            + rel).reshape(B * S)

    self_emb, agg = _sc_gather_agg(x, nodes, nb_flat, dest,
                                   B=B, S=S, R=R, D=D, CB=CB)
    return _tc_combine(self_emb, agg.reshape(B, R * D), rel, weight,
                       rel_weight, B=B, S=S, R=R, D=D, DOUT=DOUT, BB=1024)
